# baseline (speedup 1.0000x reference)
import functools

import jax
import jax.numpy as jnp
from jax import lax
from jax.experimental import pallas as pl
from jax.experimental.pallas import tpu as pltpu
from jax.experimental.pallas import tpu_sc as plsc

N_ITEMS = 819200
EMBED_DIM = 128
CACHE_SIZE = 8192

NUM_CORES = 2
NUM_SUBCORES = 16
NW = NUM_CORES * NUM_SUBCORES

B_PER_W = N_ITEMS // NW
CHUNK = 80
NG = 1
WROWS = NG * CHUNK
GROUPS = B_PER_W // WROWS
NBUF = 4
ROWS_PER_STAGER = CACHE_SIZE // NUM_SUBCORES


def _gather_body(items_hbm, table_hbm, out_hbm, idx_r,
                 rows0, rows1, rows2, rows3,
                 table_sp, isem, gsem, wsem0, wsem1, wsem2, wsem3):
    cid = lax.axis_index("c")
    sid = lax.axis_index("s")
    wid = sid * NUM_CORES + cid
    rows = (rows0, rows1, rows2, rows3)
    wsem = (wsem0, wsem1, wsem2, wsem3)

    ibase = wid * GROUPS * NG
    base = wid * B_PER_W

    pltpu.async_copy(items_hbm.at[pl.ds(ibase, NG)], idx_r.at[0], isem)
    pltpu.sync_copy(
        table_hbm.at[pl.ds(sid * ROWS_PER_STAGER, ROWS_PER_STAGER)],
        table_sp.at[pl.ds(sid * ROWS_PER_STAGER, ROWS_PER_STAGER)],
    )
    plsc.subcore_barrier()

    def super_group(sg, _):
        for b in range(NBUF):
            g = sg * NBUF + b

            pltpu.make_async_copy(
                items_hbm.at[pl.ds(ibase, NG)], idx_r.at[b], isem
            ).wait()

            @pl.when(g + 1 < GROUPS)
            def _():
                pltpu.async_copy(
                    items_hbm.at[pl.ds(ibase + (g + 1) * NG, NG)],
                    idx_r.at[(b + 1) % NBUF],
                    isem,
                )

            @pl.when(sg > 0)
            def _():
                pltpu.make_async_copy(
                    rows[b], out_hbm.at[pl.ds(base, WROWS)], wsem[b]
                ).wait()

            for k in range(NG):
                pltpu.async_copy(
                    table_sp.at[idx_r.at[b].at[k]],
                    rows[b].at[pl.ds(k * CHUNK, CHUNK)],
                    gsem,
                )
            pltpu.make_async_copy(
                table_sp.at[pl.ds(0, WROWS)], rows[b], gsem
            ).wait()
            pltpu.async_copy(
                rows[b], out_hbm.at[pl.ds(base + g * WROWS, WROWS)], wsem[b]
            )
        return ()

    lax.fori_loop(0, GROUPS // NBUF, super_group, (), unroll=False)
    for b in range(NBUF):
        pltpu.make_async_copy(
            rows[b], out_hbm.at[pl.ds(base, WROWS)], wsem[b]
        ).wait()


def _make_gather():
    mesh = plsc.VectorSubcoreMesh(core_axis_name="c", subcore_axis_name="s")
    return pl.kernel(
        _gather_body,
        mesh=mesh,
        out_type=jax.ShapeDtypeStruct((N_ITEMS, EMBED_DIM), jnp.float32),
        scratch_types=[
            pltpu.VMEM((NBUF, NG, CHUNK), jnp.int32),
            pltpu.VMEM((WROWS, EMBED_DIM), jnp.float32),
            pltpu.VMEM((WROWS, EMBED_DIM), jnp.float32),
            pltpu.VMEM((WROWS, EMBED_DIM), jnp.float32),
            pltpu.VMEM((WROWS, EMBED_DIM), jnp.float32),
            pltpu.VMEM_SHARED((CACHE_SIZE, EMBED_DIM), jnp.float32),
            pltpu.SemaphoreType.DMA,
            pltpu.SemaphoreType.DMA,
            pltpu.SemaphoreType.DMA,
            pltpu.SemaphoreType.DMA,
            pltpu.SemaphoreType.DMA,
            pltpu.SemaphoreType.DMA,
        ],
    )


_gather = _make_gather()


@jax.jit
def kernel(items, embed):
    items = items.astype(jnp.int32).reshape(NW * GROUPS * NG, CHUNK)
    embed = embed.astype(jnp.float32)
    return _gather(items, embed)


# speedup vs baseline: 1.1863x; 1.1863x over previous
import functools

import jax
import jax.numpy as jnp
from jax import lax
from jax.experimental import pallas as pl
from jax.experimental.pallas import tpu as pltpu
from jax.experimental.pallas import tpu_sc as plsc

N_ITEMS = 819200
EMBED_DIM = 128
CACHE_SIZE = 8192

NUM_CORES = 2
NUM_SUBCORES = 16
NW = NUM_CORES * NUM_SUBCORES

B_PER_W = N_ITEMS // NW
CHUNK = 100
NG = 2
WROWS = NG * CHUNK
GROUPS = B_PER_W // WROWS
NBUF = 2
ROWS_PER_STAGER = CACHE_SIZE // NUM_SUBCORES


def _gather_body(items_hbm, table_hbm, out_hbm, idx_r, rows0, rows1,
                 table_sp, isem, ssem, gsem, wsem0, wsem1):
    cid = lax.axis_index("c")
    sid = lax.axis_index("s")
    wid = sid * NUM_CORES + cid
    rows = (rows0, rows1)
    wsem = (wsem0, wsem1)

    ibase = wid * GROUPS * NG
    base = wid * B_PER_W

    pltpu.async_copy(items_hbm.at[pl.ds(ibase, NG)], idx_r.at[0], isem)
    pltpu.async_copy(
        table_hbm.at[pl.ds(sid * ROWS_PER_STAGER, ROWS_PER_STAGER)],
        table_sp.at[pl.ds(sid * ROWS_PER_STAGER, ROWS_PER_STAGER)],
        ssem,
    )

    def group_step(g, b, src, drain_write):
        pltpu.make_async_copy(
            items_hbm.at[pl.ds(ibase, NG)], idx_r.at[b], isem
        ).wait()

        def _prefetch():
            pltpu.async_copy(
                items_hbm.at[pl.ds(ibase + (g + 1) * NG, NG)],
                idx_r.at[(b + 1) % NBUF],
                isem,
            )

        if isinstance(g, int):
            if g + 1 < GROUPS:
                _prefetch()
        else:
            pl.when(g + 1 < GROUPS)(_prefetch)

        if drain_write:
            pltpu.make_async_copy(
                rows[b], out_hbm.at[pl.ds(base, WROWS)], wsem[b]
            ).wait()

        for k in range(NG):
            pltpu.async_copy(
                src.at[idx_r.at[b].at[k]],
                rows[b].at[pl.ds(k * CHUNK, CHUNK)],
                gsem,
            )
        pltpu.make_async_copy(
            table_hbm.at[pl.ds(0, WROWS)], rows[b], gsem
        ).wait()
        pltpu.async_copy(
            rows[b], out_hbm.at[pl.ds(base + g * WROWS, WROWS)], wsem[b]
        )

    for b in range(NBUF):
        group_step(b, b, table_hbm, drain_write=False)

    pltpu.make_async_copy(
        table_hbm.at[pl.ds(0, ROWS_PER_STAGER)],
        table_sp.at[pl.ds(0, ROWS_PER_STAGER)],
        ssem,
    ).wait()
    plsc.subcore_barrier()

    def super_group(sg, _):
        for b in range(NBUF):
            group_step(sg * NBUF + b, b, table_sp, drain_write=True)
        return ()

    lax.fori_loop(1, GROUPS // NBUF, super_group, (), unroll=False)
    for b in range(NBUF):
        pltpu.make_async_copy(
            rows[b], out_hbm.at[pl.ds(base, WROWS)], wsem[b]
        ).wait()


def _make_gather():
    mesh = plsc.VectorSubcoreMesh(core_axis_name="c", subcore_axis_name="s")
    return pl.kernel(
        _gather_body,
        mesh=mesh,
        out_type=jax.ShapeDtypeStruct((N_ITEMS, EMBED_DIM), jnp.float32),
        scratch_types=[
            pltpu.VMEM((NBUF, NG, CHUNK), jnp.int32),
            pltpu.VMEM((WROWS, EMBED_DIM), jnp.float32),
            pltpu.VMEM((WROWS, EMBED_DIM), jnp.float32),
            pltpu.VMEM_SHARED((CACHE_SIZE, EMBED_DIM), jnp.float32),
            pltpu.SemaphoreType.DMA,
            pltpu.SemaphoreType.DMA,
            pltpu.SemaphoreType.DMA,
            pltpu.SemaphoreType.DMA,
            pltpu.SemaphoreType.DMA,
        ],
    )


_gather = _make_gather()


@jax.jit
def kernel(items, embed):
    items = items.astype(jnp.int32).reshape(NW * GROUPS * NG, CHUNK)
    embed = embed.astype(jnp.float32)
    return _gather(items, embed)


# speedup vs baseline: 1.1901x; 1.0032x over previous
import functools

import jax
import jax.numpy as jnp
from jax import lax
from jax.experimental import pallas as pl
from jax.experimental.pallas import tpu as pltpu
from jax.experimental.pallas import tpu_sc as plsc

N_ITEMS = 819200
EMBED_DIM = 128
CACHE_SIZE = 8192

NUM_CORES = 2
NUM_SUBCORES = 16
NW = NUM_CORES * NUM_SUBCORES

B_PER_W = N_ITEMS // NW
CHUNK = 100
NG = 2
WROWS = NG * CHUNK
GROUPS = B_PER_W // WROWS
NBUF = 2
ROWS_PER_STAGER = CACHE_SIZE // NUM_SUBCORES


def _gather_body(items_hbm, table_hbm, out_hbm, idx_r, rows0, rows1,
                 table_sp, isem, ssem, gsem, wsem0, wsem1):
    cid = lax.axis_index("c")
    sid = lax.axis_index("s")
    wid = cid * NUM_SUBCORES + sid
    rows = (rows0, rows1)
    wsem = (wsem0, wsem1)

    ibase = wid * GROUPS * NG
    base = wid * B_PER_W

    pltpu.async_copy(items_hbm.at[pl.ds(ibase, NG)], idx_r.at[0], isem)
    pltpu.async_copy(
        table_hbm.at[pl.ds(sid * ROWS_PER_STAGER, ROWS_PER_STAGER)],
        table_sp.at[pl.ds(sid * ROWS_PER_STAGER, ROWS_PER_STAGER)],
        ssem,
    )

    def group_step(g, b, src, drain_write):
        pltpu.make_async_copy(
            items_hbm.at[pl.ds(ibase, NG)], idx_r.at[b], isem
        ).wait()

        def _prefetch():
            pltpu.async_copy(
                items_hbm.at[pl.ds(ibase + (g + 1) * NG, NG)],
                idx_r.at[(b + 1) % NBUF],
                isem,
            )

        if isinstance(g, int):
            if g + 1 < GROUPS:
                _prefetch()
        else:
            pl.when(g + 1 < GROUPS)(_prefetch)

        if drain_write:
            pltpu.make_async_copy(
                rows[b], out_hbm.at[pl.ds(base, WROWS)], wsem[b]
            ).wait()

        for k in range(NG):
            pltpu.async_copy(
                src.at[idx_r.at[b].at[k]],
                rows[b].at[pl.ds(k * CHUNK, CHUNK)],
                gsem,
            )
        pltpu.make_async_copy(
            table_hbm.at[pl.ds(0, WROWS)], rows[b], gsem
        ).wait()
        pltpu.async_copy(
            rows[b], out_hbm.at[pl.ds(base + g * WROWS, WROWS)], wsem[b]
        )

    for b in range(NBUF):
        group_step(b, b, table_hbm, drain_write=False)

    pltpu.make_async_copy(
        table_hbm.at[pl.ds(0, ROWS_PER_STAGER)],
        table_sp.at[pl.ds(0, ROWS_PER_STAGER)],
        ssem,
    ).wait()
    plsc.subcore_barrier()

    def super_group(sg, _):
        for b in range(NBUF):
            group_step(sg * NBUF + b, b, table_sp, drain_write=True)
        return ()

    lax.fori_loop(1, GROUPS // NBUF, super_group, (), unroll=False)
    for b in range(NBUF):
        pltpu.make_async_copy(
            rows[b], out_hbm.at[pl.ds(base, WROWS)], wsem[b]
        ).wait()


def _make_gather():
    mesh = plsc.VectorSubcoreMesh(core_axis_name="c", subcore_axis_name="s")
    return pl.kernel(
        _gather_body,
        mesh=mesh,
        out_type=jax.ShapeDtypeStruct((N_ITEMS, EMBED_DIM), jnp.float32),
        scratch_types=[
            pltpu.VMEM((NBUF, NG, CHUNK), jnp.int32),
            pltpu.VMEM((WROWS, EMBED_DIM), jnp.float32),
            pltpu.VMEM((WROWS, EMBED_DIM), jnp.float32),
            pltpu.VMEM_SHARED((CACHE_SIZE, EMBED_DIM), jnp.float32),
            pltpu.SemaphoreType.DMA,
            pltpu.SemaphoreType.DMA,
            pltpu.SemaphoreType.DMA,
            pltpu.SemaphoreType.DMA,
            pltpu.SemaphoreType.DMA,
        ],
    )


_gather = _make_gather()


@jax.jit
def kernel(items, embed):
    items = items.astype(jnp.int32).reshape(NW * GROUPS * NG, CHUNK)
    embed = embed.astype(jnp.float32)
    return _gather(items, embed)
